# baseline (speedup 1.0000x reference)
import functools

import jax
import jax.numpy as jnp
from jax import lax
from jax.experimental import pallas as pl
from jax.experimental.pallas import tpu as pltpu
from jax.experimental.pallas import tpu_sc as plsc

N_NODES = 10000
N_EDGES = 320000
D = 128

NC = 2
NS = 16
NW = NC * NS
EPT = N_EDGES // NW
CHUNK = 128
NCHUNK = ((-(-EPT // CHUNK) + 15) // 16) * 16
EPT_PAD = NCHUNK * CHUNK
HALF = NCHUNK // 2
DUMMY = N_NODES
NPAD = 10240
RPT = NPAD // NS
WB = 128
NWB = RPT // WB
LANES = 16


def _mesh():
    return plsc.VectorSubcoreMesh(core_axis_name="c", subcore_axis_name="s")


def _sc_degree(col_c, iota_c):
    @functools.partial(
        pl.kernel,
        out_type=jax.ShapeDtypeStruct((NC, NS, 8, CHUNK), jnp.float32),
        mesh=_mesh(),
        scratch_types=[
            pltpu.VMEM((NCHUNK, CHUNK), jnp.int32),
            pltpu.VMEM((NWB, CHUNK), jnp.int32),
            pltpu.VMEM((CHUNK,), jnp.float32),
            pltpu.VMEM((CHUNK,), jnp.float32),
            pltpu.VMEM((8, CHUNK), jnp.float32),
            pltpu.VMEM((CHUNK,), jnp.int32),
            pltpu.VMEM_SHARED((NPAD,), jnp.float32),
            pltpu.SemaphoreType.DMA,
        ],
    )
    def k(col_hbm, iota_hbm, out_hbm, cidx, iov, ones_b, zb, wb8, ibuf, deg_sh, sem):
        cid = lax.axis_index("c")
        sid = lax.axis_index("s")
        gid = cid * NS + sid
        pltpu.sync_copy(col_hbm.at[gid], cidx)
        pltpu.sync_copy(iota_hbm.at[sid], iov)

        ones = jnp.ones((LANES,), jnp.float32)
        zeros = jnp.zeros((LANES,), jnp.float32)
        for kk in range(CHUNK // LANES):
            ones_b[pl.ds(kk * LANES, LANES)] = ones
            zb[pl.ds(kk * LANES, LANES)] = zeros

        def load_ibuf(src_ref, j):
            for kk in range(CHUNK // LANES):
                ibuf[pl.ds(kk * LANES, LANES)] = src_ref[j, pl.ds(kk * LANES, LANES)]

        for t in range(NWB):
            load_ibuf(iov, t)
            pltpu.async_copy(zb, deg_sh.at[ibuf], sem).wait()
        plsc.subcore_barrier()

        def body(j, carry):
            load_ibuf(cidx, j)
            pltpu.async_copy(ones_b, deg_sh.at[ibuf], sem, add=True).wait()
            return carry

        lax.fori_loop(0, NCHUNK, body, 0)

        plsc.subcore_barrier()
        for t in range(NWB):
            load_ibuf(iov, t)
            pltpu.async_copy(deg_sh.at[ibuf], zb, sem).wait()
            for kk in range(CHUNK // LANES):
                wb8[t, pl.ds(kk * LANES, LANES)] = zb[pl.ds(kk * LANES, LANES)]
        pltpu.sync_copy(wb8, out_hbm.at[cid, sid])

    return k(col_c, iota_c)


def _sc_scatter(y, row_c, col_c, iota_c):
    @functools.partial(
        pl.kernel,
        out_type=jax.ShapeDtypeStruct((NC, NPAD, D), jnp.float32),
        mesh=_mesh(),
        scratch_types=[
            pltpu.VMEM((HALF, CHUNK), jnp.int32),
            pltpu.VMEM((HALF, CHUNK), jnp.int32),
            pltpu.VMEM((NWB, CHUNK), jnp.int32),
            pltpu.VMEM((CHUNK,), jnp.int32),
            pltpu.VMEM((CHUNK,), jnp.int32),
            pltpu.VMEM((CHUNK, D), jnp.float32),
            pltpu.VMEM((CHUNK, D), jnp.float32),
            pltpu.VMEM_SHARED((NPAD, D), jnp.float32),
            pltpu.SemaphoreType.DMA,
            pltpu.SemaphoreType.DMA,
            pltpu.SemaphoreType.DMA,
            pltpu.SemaphoreType.DMA,
        ],
    )
    def k(y_hbm, row_hbm, col_hbm, iota_hbm, out_hbm,
          ridx, cidx, iov, ibuf, ibuf1, gb0, gb1, acc, s0, s1, s2, s3):
        cid = lax.axis_index("c")
        sid = lax.axis_index("s")
        gid = cid * NS + sid
        pltpu.sync_copy(iota_hbm.at[sid], iov)

        def load_ibuf(src_ref, j):
            for kk in range(CHUNK // LANES):
                ibuf[pl.ds(kk * LANES, LANES)] = src_ref[j, pl.ds(kk * LANES, LANES)]

        def load_ibuf1(src_ref, j):
            for kk in range(CHUNK // LANES):
                ibuf1[pl.ds(kk * LANES, LANES)] = src_ref[j, pl.ds(kk * LANES, LANES)]

        def zero(i, carry):
            for kk in range(D // LANES):
                gb0[i, pl.ds(kk * LANES, LANES)] = jnp.zeros((LANES,), jnp.float32)
            return carry

        lax.fori_loop(0, CHUNK, zero, 0)

        base = sid * RPT
        for t in range(NWB):
            load_ibuf(iov, t)
            pltpu.async_copy(gb0, acc.at[ibuf], s0).wait()
        plsc.subcore_barrier()

        def gstart(j, buf, sem):
            pltpu.make_async_copy(y_hbm.at[ridx.at[j]], buf, sem).start()

        def gwait(j, buf, sem):
            pltpu.make_async_copy(y_hbm.at[ridx.at[j]], buf, sem).wait()

        def sstart(buf, ib, sem):
            pltpu.make_async_copy(buf, acc.at[ib], sem).start(add=True)

        def swait(buf, ib, sem):
            pltpu.make_async_copy(buf, acc.at[ib], sem).wait()

        for p in range(NCHUNK // HALF):
            pltpu.sync_copy(row_hbm.at[gid, pl.ds(p * HALF, HALF)], ridx)
            pltpu.sync_copy(col_hbm.at[gid, pl.ds(p * HALF, HALF)], cidx)

            gstart(0, gb0, s0)
            gstart(1, gb1, s2)

            def body(i, carry):
                j = 2 * i
                gwait(j, gb0, s0)
                load_ibuf(cidx, j)
                sstart(gb0, ibuf, s1)
                gwait(j + 1, gb1, s2)
                load_ibuf1(cidx, j + 1)
                sstart(gb1, ibuf1, s3)
                swait(gb0, ibuf, s1)
                gstart(j + 2, gb0, s0)
                swait(gb1, ibuf1, s3)
                gstart(j + 3, gb1, s2)
                return carry

            lax.fori_loop(0, HALF // 2 - 1, body, 0)
            jt = HALF - 2
            gwait(jt, gb0, s0)
            load_ibuf(cidx, jt)
            sstart(gb0, ibuf, s1)
            gwait(jt + 1, gb1, s2)
            load_ibuf1(cidx, jt + 1)
            sstart(gb1, ibuf1, s3)
            swait(gb0, ibuf, s1)
            swait(gb1, ibuf1, s3)

        plsc.subcore_barrier()
        for t in range(NWB):
            r = base + t * WB
            load_ibuf(iov, t)
            pltpu.async_copy(acc.at[ibuf], gb0, s0).wait()
            pltpu.sync_copy(gb0, out_hbm.at[cid, pl.ds(r, WB)])

    return k(y, row_c, col_c, iota_c)


_R = 1000
_G = N_NODES // _R


def _dinv_block(d0_ref, d1_ref):
    deg = d0_ref[...][0] + d1_ref[...][0] + 1.0
    return lax.rsqrt(deg)


def _deg_specs():
    return [
        pl.BlockSpec((1, _R, 1), lambda i: (0, i, 0)),
        pl.BlockSpec((1, _R, 1), lambda i: (1, i, 0)),
    ]


def _ap_specs():
    return [
        pl.BlockSpec((1, _R, D), lambda i: (0, i, 0)),
        pl.BlockSpec((1, _R, D), lambda i: (1, i, 0)),
    ]


def _tc_mm1(x, w1, degp):
    def body(x_ref, w_ref, d0_ref, d1_ref, y_ref):
        dinv = _dinv_block(d0_ref, d1_ref)
        y_ref[...] = jnp.dot(x_ref[...], w_ref[...],
                             preferred_element_type=jnp.float32) * dinv

    return pl.pallas_call(
        body,
        grid=(_G,),
        in_specs=[
            pl.BlockSpec((_R, D), lambda i: (i, 0)),
            pl.BlockSpec((D, D), lambda i: (0, 0)),
        ] + _deg_specs(),
        out_specs=pl.BlockSpec((_R, D), lambda i: (i, 0)),
        out_shape=jax.ShapeDtypeStruct((N_NODES, D), jnp.float32),
    )(x, w1, degp, degp)


def _tc_mm2(ap, y1, degp, b1, w2):
    def body(a0_ref, a1_ref, y1_ref, d0_ref, d1_ref, b_ref, w_ref, y2_ref):
        dinv = _dinv_block(d0_ref, d1_ref)
        h = (a0_ref[...][0] + a1_ref[...][0] + y1_ref[...]) * dinv + b_ref[...]
        h = jnp.maximum(h, 0.0)
        y2_ref[...] = jnp.dot(h, w_ref[...],
                              preferred_element_type=jnp.float32) * dinv

    return pl.pallas_call(
        body,
        grid=(_G,),
        in_specs=_ap_specs() + [
            pl.BlockSpec((_R, D), lambda i: (i, 0)),
        ] + _deg_specs() + [
            pl.BlockSpec((1, D), lambda i: (0, 0)),
            pl.BlockSpec((D, D), lambda i: (0, 0)),
        ],
        out_specs=pl.BlockSpec((_R, D), lambda i: (i, 0)),
        out_shape=jax.ShapeDtypeStruct((N_NODES, D), jnp.float32),
    )(ap, ap, y1, degp, degp, b1, w2)


def _tc_final(ap, y2, degp, b2):
    def body(a0_ref, a1_ref, y2_ref, d0_ref, d1_ref, b_ref, o_ref):
        dinv = _dinv_block(d0_ref, d1_ref)
        h = (a0_ref[...][0] + a1_ref[...][0] + y2_ref[...]) * dinv + b_ref[...]
        o_ref[...] = jnp.maximum(h, 0.0)

    return pl.pallas_call(
        body,
        grid=(_G,),
        in_specs=_ap_specs() + [
            pl.BlockSpec((_R, D), lambda i: (i, 0)),
        ] + _deg_specs() + [
            pl.BlockSpec((1, D), lambda i: (0, 0)),
        ],
        out_specs=pl.BlockSpec((_R, D), lambda i: (i, 0)),
        out_shape=jax.ShapeDtypeStruct((N_NODES, D), jnp.float32),
    )(ap, ap, y2, degp, degp, b2)


def kernel(x, edge_index, W1, b1, W2, b2):
    ei = edge_index.astype(jnp.int32)
    row = ei[0].reshape(NW, EPT)
    col = ei[1].reshape(NW, EPT)
    npad = EPT_PAD - EPT
    row_c = jnp.concatenate(
        [row, jnp.zeros((NW, npad), jnp.int32)], axis=1).reshape(NW, NCHUNK, CHUNK)
    col_c = jnp.concatenate(
        [col, jnp.full((NW, npad), DUMMY, jnp.int32)], axis=1).reshape(NW, NCHUNK, CHUNK)

    iota_c = jnp.arange(NPAD, dtype=jnp.int32).reshape(NS, NWB, CHUNK)
    degr = _sc_degree(col_c, iota_c)
    degp = degr[:, :, :NWB, :].reshape(NC, NPAD, 1)
    b1r = b1.reshape(1, D)
    b2r = b2.reshape(1, D)

    y1 = _tc_mm1(x, W1, degp)
    a1 = _sc_scatter(y1, row_c, col_c, iota_c)
    y2 = _tc_mm2(a1, y1, degp, b1r, W2)
    a2 = _sc_scatter(y2, row_c, col_c, iota_c)
    return _tc_final(a2, y2, degp, b2r)


# speedup vs baseline: 1.0751x; 1.0751x over previous
import functools

import jax
import jax.numpy as jnp
from jax import lax
from jax.experimental import pallas as pl
from jax.experimental.pallas import tpu as pltpu
from jax.experimental.pallas import tpu_sc as plsc

N_NODES = 10000
N_EDGES = 320000
D = 128

NC = 2
NS = 16
NW = NC * NS
EPT = N_EDGES // NW
CHUNK = 128
NCHUNK = ((-(-EPT // CHUNK) + 15) // 16) * 16
EPT_PAD = NCHUNK * CHUNK
HALF = NCHUNK // 2
DUMMY = N_NODES
NPAD = 10240
RPT = NPAD // NS
WB = 128
NWB = RPT // WB
LANES = 16


def _mesh():
    return plsc.VectorSubcoreMesh(core_axis_name="c", subcore_axis_name="s")


def _sc_degree(col_c, iota_c):
    @functools.partial(
        pl.kernel,
        out_type=jax.ShapeDtypeStruct((NC, NS, 8, CHUNK), jnp.float32),
        mesh=_mesh(),
        scratch_types=[
            pltpu.VMEM((NCHUNK, CHUNK), jnp.int32),
            pltpu.VMEM((NWB, CHUNK), jnp.int32),
            pltpu.VMEM((CHUNK,), jnp.float32),
            pltpu.VMEM((CHUNK,), jnp.float32),
            pltpu.VMEM((8, CHUNK), jnp.float32),
            pltpu.VMEM((CHUNK,), jnp.int32),
            pltpu.VMEM_SHARED((NPAD,), jnp.float32),
            pltpu.SemaphoreType.DMA,
        ],
    )
    def k(col_hbm, iota_hbm, out_hbm, cidx, iov, ones_b, zb, wb8, ibuf, deg_sh, sem):
        cid = lax.axis_index("c")
        sid = lax.axis_index("s")
        gid = cid * NS + sid
        pltpu.sync_copy(col_hbm.at[gid], cidx)
        pltpu.sync_copy(iota_hbm.at[sid], iov)

        ones = jnp.ones((LANES,), jnp.float32)
        zeros = jnp.zeros((LANES,), jnp.float32)
        for kk in range(CHUNK // LANES):
            ones_b[pl.ds(kk * LANES, LANES)] = ones
            zb[pl.ds(kk * LANES, LANES)] = zeros

        def load_ibuf(src_ref, j):
            for kk in range(CHUNK // LANES):
                ibuf[pl.ds(kk * LANES, LANES)] = src_ref[j, pl.ds(kk * LANES, LANES)]

        for t in range(NWB):
            load_ibuf(iov, t)
            pltpu.async_copy(zb, deg_sh.at[ibuf], sem).wait()
        plsc.subcore_barrier()

        def body(j, carry):
            load_ibuf(cidx, j)
            pltpu.async_copy(ones_b, deg_sh.at[ibuf], sem, add=True).wait()
            return carry

        lax.fori_loop(0, NCHUNK, body, 0)

        plsc.subcore_barrier()
        for t in range(NWB):
            load_ibuf(iov, t)
            pltpu.async_copy(deg_sh.at[ibuf], zb, sem).wait()
            for kk in range(CHUNK // LANES):
                wb8[t, pl.ds(kk * LANES, LANES)] = zb[pl.ds(kk * LANES, LANES)]
        pltpu.sync_copy(wb8, out_hbm.at[cid, sid])

    return k(col_c, iota_c)


def _sc_scatter(y, row_c, col_c, iota_c):
    @functools.partial(
        pl.kernel,
        out_type=jax.ShapeDtypeStruct((NC, NPAD, D), jnp.float32),
        mesh=_mesh(),
        scratch_types=[
            pltpu.VMEM((HALF, CHUNK), jnp.int32),
            pltpu.VMEM((HALF, CHUNK), jnp.int32),
            pltpu.VMEM((NWB, CHUNK), jnp.int32),
            pltpu.VMEM((CHUNK,), jnp.int32),
            pltpu.VMEM((CHUNK,), jnp.int32),
            pltpu.VMEM((CHUNK, D), jnp.float32),
            pltpu.VMEM((CHUNK, D), jnp.float32),
            pltpu.VMEM_SHARED((NPAD, D), jnp.float32),
            pltpu.SemaphoreType.DMA,
            pltpu.SemaphoreType.DMA,
            pltpu.SemaphoreType.DMA,
            pltpu.SemaphoreType.DMA,
        ],
    )
    def k(y_hbm, row_hbm, col_hbm, iota_hbm, out_hbm,
          ridx, cidx, iov, ibuf, ibuf1, gb0, gb1, acc, s0, s1, s2, s3):
        cid = lax.axis_index("c")
        sid = lax.axis_index("s")
        gid = cid * NS + sid
        pltpu.sync_copy(iota_hbm.at[sid], iov)

        def load_ibuf(src_ref, j):
            for kk in range(CHUNK // LANES):
                ibuf[pl.ds(kk * LANES, LANES)] = src_ref[j, pl.ds(kk * LANES, LANES)]

        def zero(i, carry):
            for kk in range(D // LANES):
                gb0[i, pl.ds(kk * LANES, LANES)] = jnp.zeros((LANES,), jnp.float32)
            return carry

        lax.fori_loop(0, CHUNK, zero, 0)

        base = sid * RPT
        for t in range(NWB):
            load_ibuf(iov, t)
            pltpu.async_copy(gb0, acc.at[ibuf], s0).wait()
        plsc.subcore_barrier()

        def gstart(j, buf, sem):
            pltpu.make_async_copy(y_hbm.at[ridx.at[j]], buf, sem).start()

        def gwait(j, buf, sem):
            pltpu.make_async_copy(y_hbm.at[ridx.at[j]], buf, sem).wait()

        def scat(j, buf):
            load_ibuf(cidx, j)
            pltpu.async_copy(buf, acc.at[ibuf], s1, add=True).wait()

        for p in range(NCHUNK // HALF):
            pltpu.sync_copy(row_hbm.at[gid, pl.ds(p * HALF, HALF)], ridx)
            pltpu.sync_copy(col_hbm.at[gid, pl.ds(p * HALF, HALF)], cidx)

            gstart(0, gb0, s0)
            gstart(1, gb1, s2)

            def body(i, carry):
                j = 2 * i
                gwait(j, gb0, s0)
                scat(j, gb0)
                gstart(j + 2, gb0, s0)
                gwait(j + 1, gb1, s2)
                scat(j + 1, gb1)
                gstart(j + 3, gb1, s2)
                return carry

            lax.fori_loop(0, HALF // 2 - 1, body, 0)
            jt = HALF - 2
            gwait(jt, gb0, s0)
            scat(jt, gb0)
            gwait(jt + 1, gb1, s2)
            scat(jt + 1, gb1)

        plsc.subcore_barrier()
        for t in range(NWB):
            r = base + t * WB
            load_ibuf(iov, t)
            pltpu.async_copy(acc.at[ibuf], gb0, s0).wait()
            pltpu.sync_copy(gb0, out_hbm.at[cid, pl.ds(r, WB)])

    return k(y, row_c, col_c, iota_c)


_R = 1000
_G = N_NODES // _R


def _dinv_block(d0_ref, d1_ref):
    deg = d0_ref[...][0] + d1_ref[...][0] + 1.0
    return lax.rsqrt(deg)


def _deg_specs():
    return [
        pl.BlockSpec((1, _R, 1), lambda i: (0, i, 0)),
        pl.BlockSpec((1, _R, 1), lambda i: (1, i, 0)),
    ]


def _ap_specs():
    return [
        pl.BlockSpec((1, _R, D), lambda i: (0, i, 0)),
        pl.BlockSpec((1, _R, D), lambda i: (1, i, 0)),
    ]


def _tc_mm1(x, w1, degp):
    def body(x_ref, w_ref, d0_ref, d1_ref, y_ref):
        dinv = _dinv_block(d0_ref, d1_ref)
        y_ref[...] = jnp.dot(x_ref[...], w_ref[...],
                             preferred_element_type=jnp.float32) * dinv

    return pl.pallas_call(
        body,
        grid=(_G,),
        in_specs=[
            pl.BlockSpec((_R, D), lambda i: (i, 0)),
            pl.BlockSpec((D, D), lambda i: (0, 0)),
        ] + _deg_specs(),
        out_specs=pl.BlockSpec((_R, D), lambda i: (i, 0)),
        out_shape=jax.ShapeDtypeStruct((N_NODES, D), jnp.float32),
    )(x, w1, degp, degp)


def _tc_mm2(ap, y1, degp, b1, w2):
    def body(a0_ref, a1_ref, y1_ref, d0_ref, d1_ref, b_ref, w_ref, y2_ref):
        dinv = _dinv_block(d0_ref, d1_ref)
        h = (a0_ref[...][0] + a1_ref[...][0] + y1_ref[...]) * dinv + b_ref[...]
        h = jnp.maximum(h, 0.0)
        y2_ref[...] = jnp.dot(h, w_ref[...],
                              preferred_element_type=jnp.float32) * dinv

    return pl.pallas_call(
        body,
        grid=(_G,),
        in_specs=_ap_specs() + [
            pl.BlockSpec((_R, D), lambda i: (i, 0)),
        ] + _deg_specs() + [
            pl.BlockSpec((1, D), lambda i: (0, 0)),
            pl.BlockSpec((D, D), lambda i: (0, 0)),
        ],
        out_specs=pl.BlockSpec((_R, D), lambda i: (i, 0)),
        out_shape=jax.ShapeDtypeStruct((N_NODES, D), jnp.float32),
    )(ap, ap, y1, degp, degp, b1, w2)


def _tc_final(ap, y2, degp, b2):
    def body(a0_ref, a1_ref, y2_ref, d0_ref, d1_ref, b_ref, o_ref):
        dinv = _dinv_block(d0_ref, d1_ref)
        h = (a0_ref[...][0] + a1_ref[...][0] + y2_ref[...]) * dinv + b_ref[...]
        o_ref[...] = jnp.maximum(h, 0.0)

    return pl.pallas_call(
        body,
        grid=(_G,),
        in_specs=_ap_specs() + [
            pl.BlockSpec((_R, D), lambda i: (i, 0)),
        ] + _deg_specs() + [
            pl.BlockSpec((1, D), lambda i: (0, 0)),
        ],
        out_specs=pl.BlockSpec((_R, D), lambda i: (i, 0)),
        out_shape=jax.ShapeDtypeStruct((N_NODES, D), jnp.float32),
    )(ap, ap, y2, degp, degp, b2)


def kernel(x, edge_index, W1, b1, W2, b2):
    ei = edge_index.astype(jnp.int32)
    row = ei[0].reshape(NW, EPT)
    col = ei[1].reshape(NW, EPT)
    npad = EPT_PAD - EPT
    row_c = jnp.concatenate(
        [row, jnp.zeros((NW, npad), jnp.int32)], axis=1).reshape(NW, NCHUNK, CHUNK)
    col_c = jnp.concatenate(
        [col, jnp.full((NW, npad), DUMMY, jnp.int32)], axis=1).reshape(NW, NCHUNK, CHUNK)

    iota_c = jnp.arange(NPAD, dtype=jnp.int32).reshape(NS, NWB, CHUNK)
    degr = _sc_degree(col_c, iota_c)
    degp = degr[:, :, :NWB, :].reshape(NC, NPAD, 1)
    b1r = b1.reshape(1, D)
    b2r = b2.reshape(1, D)

    y1 = _tc_mm1(x, W1, degp)
    a1 = _sc_scatter(y1, row_c, col_c, iota_c)
    y2 = _tc_mm2(a1, y1, degp, b1r, W2)
    a2 = _sc_scatter(y2, row_c, col_c, iota_c)
    return _tc_final(a2, y2, degp, b2r)


# speedup vs baseline: 1.0797x; 1.0043x over previous
import functools

import jax
import jax.numpy as jnp
from jax import lax
from jax.experimental import pallas as pl
from jax.experimental.pallas import tpu as pltpu
from jax.experimental.pallas import tpu_sc as plsc

N_NODES = 10000
N_EDGES = 320000
D = 128

NC = 2
NS = 16
NW = NC * NS
EPT = N_EDGES // NW
CHUNK = 128
NCHUNK = ((-(-EPT // CHUNK) + 15) // 16) * 16
EPT_PAD = NCHUNK * CHUNK
HALF = NCHUNK // 2
DUMMY = N_NODES
NPAD = 10240
RPT = NPAD // NS
WB = 128
NWB = RPT // WB
LANES = 16


def _mesh():
    return plsc.VectorSubcoreMesh(core_axis_name="c", subcore_axis_name="s")


def _sc_degree(col_c, iota_c):
    @functools.partial(
        pl.kernel,
        out_type=jax.ShapeDtypeStruct((NC, NS, 8, CHUNK), jnp.float32),
        mesh=_mesh(),
        scratch_types=[
            pltpu.VMEM((NCHUNK, CHUNK), jnp.int32),
            pltpu.VMEM((NWB, CHUNK), jnp.int32),
            pltpu.VMEM((CHUNK,), jnp.float32),
            pltpu.VMEM((CHUNK,), jnp.float32),
            pltpu.VMEM((8, CHUNK), jnp.float32),
            pltpu.VMEM((CHUNK,), jnp.int32),
            pltpu.VMEM_SHARED((NPAD,), jnp.float32),
            pltpu.SemaphoreType.DMA,
        ],
    )
    def k(col_hbm, iota_hbm, out_hbm, cidx, iov, ones_b, zb, wb8, ibuf, deg_sh, sem):
        cid = lax.axis_index("c")
        sid = lax.axis_index("s")
        gid = cid * NS + sid
        pltpu.sync_copy(col_hbm.at[gid], cidx)
        pltpu.sync_copy(iota_hbm.at[sid], iov)

        ones = jnp.ones((LANES,), jnp.float32)
        zeros = jnp.zeros((LANES,), jnp.float32)
        for kk in range(CHUNK // LANES):
            ones_b[pl.ds(kk * LANES, LANES)] = ones
            zb[pl.ds(kk * LANES, LANES)] = zeros

        def load_ibuf(src_ref, j):
            for kk in range(CHUNK // LANES):
                ibuf[pl.ds(kk * LANES, LANES)] = src_ref[j, pl.ds(kk * LANES, LANES)]

        for t in range(NWB):
            load_ibuf(iov, t)
            pltpu.async_copy(zb, deg_sh.at[ibuf], sem).wait()
        plsc.subcore_barrier()

        def body(j, carry):
            load_ibuf(cidx, j)
            pltpu.async_copy(ones_b, deg_sh.at[ibuf], sem, add=True).wait()
            return carry

        lax.fori_loop(0, NCHUNK, body, 0)

        plsc.subcore_barrier()
        for t in range(NWB):
            load_ibuf(iov, t)
            pltpu.async_copy(deg_sh.at[ibuf], zb, sem).wait()
            for kk in range(CHUNK // LANES):
                wb8[t, pl.ds(kk * LANES, LANES)] = zb[pl.ds(kk * LANES, LANES)]
        pltpu.sync_copy(wb8, out_hbm.at[cid, sid])

    return k(col_c, iota_c)


def _sc_scatter(y, row_c, col_c, iota_c):
    @functools.partial(
        pl.kernel,
        out_type=jax.ShapeDtypeStruct((NC, NPAD, D), jnp.float32),
        mesh=_mesh(),
        scratch_types=[
            pltpu.VMEM((HALF, CHUNK), jnp.int32),
            pltpu.VMEM((HALF, CHUNK), jnp.int32),
            pltpu.VMEM((NWB, CHUNK), jnp.int32),
            pltpu.VMEM((CHUNK,), jnp.int32),
            pltpu.VMEM((CHUNK,), jnp.int32),
            pltpu.VMEM((CHUNK, D), jnp.float32),
            pltpu.VMEM((CHUNK, D), jnp.float32),
            pltpu.VMEM_SHARED((NPAD, D), jnp.float32),
            pltpu.SemaphoreType.DMA,
            pltpu.SemaphoreType.DMA,
            pltpu.SemaphoreType.DMA,
            pltpu.SemaphoreType.DMA,
        ],
    )
    def k(y_hbm, row_hbm, col_hbm, iota_hbm, out_hbm,
          ridx, cidx, iov, ibuf, ibuf1, gb0, gb1, acc, s0, s1, s2, s3):
        cid = lax.axis_index("c")
        sid = lax.axis_index("s")
        gid = cid * NS + sid
        pltpu.sync_copy(iota_hbm.at[sid], iov)

        def load_ibuf(src_ref, j):
            for kk in range(CHUNK // LANES):
                ibuf[pl.ds(kk * LANES, LANES)] = src_ref[j, pl.ds(kk * LANES, LANES)]

        def zero(i, carry):
            for kk in range(D // LANES):
                gb0[i, pl.ds(kk * LANES, LANES)] = jnp.zeros((LANES,), jnp.float32)
            return carry

        lax.fori_loop(0, CHUNK, zero, 0)

        base = sid * RPT
        for t in range(NWB):
            load_ibuf(iov, t)
            pltpu.async_copy(gb0, acc.at[ibuf], s0).wait()
        plsc.subcore_barrier()

        def gstart(j, buf, sem):
            pltpu.make_async_copy(y_hbm.at[ridx.at[j]], buf, sem).start()

        def gwait(j, buf, sem):
            pltpu.make_async_copy(y_hbm.at[ridx.at[j]], buf, sem).wait()

        def scat(j, buf):
            load_ibuf(cidx, j)
            pltpu.async_copy(buf, acc.at[ibuf], s1, add=True).wait()

        for p in range(NCHUNK // HALF):
            pltpu.sync_copy(row_hbm.at[gid, pl.ds(p * HALF, HALF)], ridx)
            pltpu.sync_copy(col_hbm.at[gid, pl.ds(p * HALF, HALF)], cidx)

            gstart(0, gb0, s0)
            gstart(1, gb1, s2)

            def body(i, carry):
                j = 2 * i
                gwait(j, gb0, s0)
                scat(j, gb0)
                gstart(j + 2, gb0, s0)
                gwait(j + 1, gb1, s2)
                scat(j + 1, gb1)
                gstart(j + 3, gb1, s2)
                return carry

            lax.fori_loop(0, HALF // 2 - 1, body, 0)
            jt = HALF - 2
            gwait(jt, gb0, s0)
            scat(jt, gb0)
            gwait(jt + 1, gb1, s2)
            scat(jt + 1, gb1)

        plsc.subcore_barrier()

        def wb_start(t, buf, sem):
            pltpu.make_async_copy(acc.at[iov.at[t]], buf, sem).start()

        def wb_wait(t, buf, sem):
            pltpu.make_async_copy(acc.at[iov.at[t]], buf, sem).wait()

        wb_start(0, gb0, s0)
        for t in range(NWB):
            buf, sem = (gb0, s0) if t % 2 == 0 else (gb1, s2)
            nbuf, nsem = (gb1, s2) if t % 2 == 0 else (gb0, s0)
            wb_wait(t, buf, sem)
            if t + 1 < NWB:
                wb_start(t + 1, nbuf, nsem)
            pltpu.sync_copy(buf, out_hbm.at[cid, pl.ds(base + t * WB, WB)])

    return k(y, row_c, col_c, iota_c)


_R = 1000
_G = N_NODES // _R


def _dinv_block(d0_ref, d1_ref):
    deg = d0_ref[...][0] + d1_ref[...][0] + 1.0
    return lax.rsqrt(deg)


def _deg_specs():
    return [
        pl.BlockSpec((1, _R, 1), lambda i: (0, i, 0)),
        pl.BlockSpec((1, _R, 1), lambda i: (1, i, 0)),
    ]


def _ap_specs():
    return [
        pl.BlockSpec((1, _R, D), lambda i: (0, i, 0)),
        pl.BlockSpec((1, _R, D), lambda i: (1, i, 0)),
    ]


def _tc_mm1(x, w1, degp):
    def body(x_ref, w_ref, d0_ref, d1_ref, y_ref):
        dinv = _dinv_block(d0_ref, d1_ref)
        y_ref[...] = jnp.dot(x_ref[...], w_ref[...],
                             preferred_element_type=jnp.float32) * dinv

    return pl.pallas_call(
        body,
        grid=(_G,),
        in_specs=[
            pl.BlockSpec((_R, D), lambda i: (i, 0)),
            pl.BlockSpec((D, D), lambda i: (0, 0)),
        ] + _deg_specs(),
        out_specs=pl.BlockSpec((_R, D), lambda i: (i, 0)),
        out_shape=jax.ShapeDtypeStruct((N_NODES, D), jnp.float32),
    )(x, w1, degp, degp)


def _tc_mm2(ap, y1, degp, b1, w2):
    def body(a0_ref, a1_ref, y1_ref, d0_ref, d1_ref, b_ref, w_ref, y2_ref):
        dinv = _dinv_block(d0_ref, d1_ref)
        h = (a0_ref[...][0] + a1_ref[...][0] + y1_ref[...]) * dinv + b_ref[...]
        h = jnp.maximum(h, 0.0)
        y2_ref[...] = jnp.dot(h, w_ref[...],
                              preferred_element_type=jnp.float32) * dinv

    return pl.pallas_call(
        body,
        grid=(_G,),
        in_specs=_ap_specs() + [
            pl.BlockSpec((_R, D), lambda i: (i, 0)),
        ] + _deg_specs() + [
            pl.BlockSpec((1, D), lambda i: (0, 0)),
            pl.BlockSpec((D, D), lambda i: (0, 0)),
        ],
        out_specs=pl.BlockSpec((_R, D), lambda i: (i, 0)),
        out_shape=jax.ShapeDtypeStruct((N_NODES, D), jnp.float32),
    )(ap, ap, y1, degp, degp, b1, w2)


def _tc_final(ap, y2, degp, b2):
    def body(a0_ref, a1_ref, y2_ref, d0_ref, d1_ref, b_ref, o_ref):
        dinv = _dinv_block(d0_ref, d1_ref)
        h = (a0_ref[...][0] + a1_ref[...][0] + y2_ref[...]) * dinv + b_ref[...]
        o_ref[...] = jnp.maximum(h, 0.0)

    return pl.pallas_call(
        body,
        grid=(_G,),
        in_specs=_ap_specs() + [
            pl.BlockSpec((_R, D), lambda i: (i, 0)),
        ] + _deg_specs() + [
            pl.BlockSpec((1, D), lambda i: (0, 0)),
        ],
        out_specs=pl.BlockSpec((_R, D), lambda i: (i, 0)),
        out_shape=jax.ShapeDtypeStruct((N_NODES, D), jnp.float32),
    )(ap, ap, y2, degp, degp, b2)


def kernel(x, edge_index, W1, b1, W2, b2):
    ei = edge_index.astype(jnp.int32)
    row = ei[0].reshape(NW, EPT)
    col = ei[1].reshape(NW, EPT)
    npad = EPT_PAD - EPT
    row_c = jnp.concatenate(
        [row, jnp.zeros((NW, npad), jnp.int32)], axis=1).reshape(NW, NCHUNK, CHUNK)
    col_c = jnp.concatenate(
        [col, jnp.full((NW, npad), DUMMY, jnp.int32)], axis=1).reshape(NW, NCHUNK, CHUNK)

    iota_c = jnp.arange(NPAD, dtype=jnp.int32).reshape(NS, NWB, CHUNK)
    degr = _sc_degree(col_c, iota_c)
    degp = degr[:, :, :NWB, :].reshape(NC, NPAD, 1)
    b1r = b1.reshape(1, D)
    b2r = b2.reshape(1, D)

    y1 = _tc_mm1(x, W1, degp)
    a1 = _sc_scatter(y1, row_c, col_c, iota_c)
    y2 = _tc_mm2(a1, y1, degp, b1r, W2)
    a2 = _sc_scatter(y2, row_c, col_c, iota_c)
    return _tc_final(a2, y2, degp, b2r)


# speedup vs baseline: 1.0810x; 1.0012x over previous
import functools

import jax
import jax.numpy as jnp
from jax import lax
from jax.experimental import pallas as pl
from jax.experimental.pallas import tpu as pltpu
from jax.experimental.pallas import tpu_sc as plsc

N_NODES = 10000
N_EDGES = 320000
D = 128

NC = 2
NS = 16
NW = NC * NS
EPT = N_EDGES // NW
CHUNK = 128
NCHUNK = ((-(-EPT // CHUNK) + 15) // 16) * 16
EPT_PAD = NCHUNK * CHUNK
HALF = NCHUNK // 2
DUMMY = N_NODES
NPAD = 10240
RPT = NPAD // NS
WB = 128
NWB = RPT // WB
LANES = 16


def _mesh():
    return plsc.VectorSubcoreMesh(core_axis_name="c", subcore_axis_name="s")


def _sc_degree(col_c, iota_c):
    @functools.partial(
        pl.kernel,
        out_type=jax.ShapeDtypeStruct((NC, NS, 8, CHUNK), jnp.float32),
        mesh=_mesh(),
        scratch_types=[
            pltpu.VMEM((NCHUNK, CHUNK), jnp.int32),
            pltpu.VMEM((NWB, CHUNK), jnp.int32),
            pltpu.VMEM((CHUNK,), jnp.float32),
            pltpu.VMEM((CHUNK,), jnp.float32),
            pltpu.VMEM((8, CHUNK), jnp.float32),
            pltpu.VMEM((CHUNK,), jnp.int32),
            pltpu.VMEM_SHARED((NPAD,), jnp.float32),
            pltpu.SemaphoreType.DMA,
        ],
    )
    def k(col_hbm, iota_hbm, out_hbm, cidx, iov, ones_b, zb, wb8, ibuf, deg_sh, sem):
        cid = lax.axis_index("c")
        sid = lax.axis_index("s")
        gid = cid * NS + sid
        pltpu.sync_copy(col_hbm.at[gid], cidx)
        pltpu.sync_copy(iota_hbm.at[sid], iov)

        ones = jnp.ones((LANES,), jnp.float32)
        zeros = jnp.zeros((LANES,), jnp.float32)
        for kk in range(CHUNK // LANES):
            ones_b[pl.ds(kk * LANES, LANES)] = ones
            zb[pl.ds(kk * LANES, LANES)] = zeros

        def load_ibuf(src_ref, j):
            for kk in range(CHUNK // LANES):
                ibuf[pl.ds(kk * LANES, LANES)] = src_ref[j, pl.ds(kk * LANES, LANES)]

        for t in range(NWB):
            load_ibuf(iov, t)
            pltpu.async_copy(zb, deg_sh.at[ibuf], sem).wait()
        plsc.subcore_barrier()

        def body(j, carry):
            load_ibuf(cidx, j)
            pltpu.async_copy(ones_b, deg_sh.at[ibuf], sem, add=True).wait()
            return carry

        lax.fori_loop(0, NCHUNK, body, 0)

        plsc.subcore_barrier()
        for t in range(NWB):
            load_ibuf(iov, t)
            pltpu.async_copy(deg_sh.at[ibuf], zb, sem).wait()
            for kk in range(CHUNK // LANES):
                wb8[t, pl.ds(kk * LANES, LANES)] = zb[pl.ds(kk * LANES, LANES)]
        pltpu.sync_copy(wb8, out_hbm.at[cid, sid])

    return k(col_c, iota_c)


def _sc_scatter(y, row_c, col_c, iota_c):
    @functools.partial(
        pl.kernel,
        out_type=jax.ShapeDtypeStruct((NC, NPAD, D), jnp.float32),
        mesh=_mesh(),
        scratch_types=[
            pltpu.VMEM((HALF, CHUNK), jnp.int32),
            pltpu.VMEM((HALF, CHUNK), jnp.int32),
            pltpu.VMEM((NWB, CHUNK), jnp.int32),
            pltpu.VMEM((CHUNK,), jnp.int32),
            pltpu.VMEM((CHUNK,), jnp.int32),
            pltpu.VMEM((CHUNK, D), jnp.float32),
            pltpu.VMEM((CHUNK, D), jnp.float32),
            pltpu.VMEM_SHARED((NPAD, D), jnp.float32),
            pltpu.SemaphoreType.DMA,
            pltpu.SemaphoreType.DMA,
            pltpu.SemaphoreType.DMA,
            pltpu.SemaphoreType.DMA,
        ],
    )
    def k(y_hbm, row_hbm, col_hbm, iota_hbm, out_hbm,
          ridx, cidx, iov, ibuf, ibuf1, gb0, gb1, acc, s0, s1, s2, s3):
        cid = lax.axis_index("c")
        sid = lax.axis_index("s")
        gid = cid * NS + sid
        pltpu.sync_copy(iota_hbm.at[sid], iov)

        def load_ibuf(src_ref, j):
            for kk in range(CHUNK // LANES):
                ibuf[pl.ds(kk * LANES, LANES)] = src_ref[j, pl.ds(kk * LANES, LANES)]

        def zero(i, carry):
            for kk in range(D // LANES):
                gb0[i, pl.ds(kk * LANES, LANES)] = jnp.zeros((LANES,), jnp.float32)
            return carry

        lax.fori_loop(0, CHUNK, zero, 0)

        base = sid * RPT
        for t in range(NWB):
            load_ibuf(iov, t)
            pltpu.async_copy(gb0, acc.at[ibuf], s0).wait()
        plsc.subcore_barrier()

        def gstart(j, buf, sem):
            pltpu.make_async_copy(y_hbm.at[ridx.at[j]], buf, sem).start()

        def gwait(j, buf, sem):
            pltpu.make_async_copy(y_hbm.at[ridx.at[j]], buf, sem).wait()

        def scat(j, buf):
            pltpu.async_copy(buf, acc.at[cidx.at[j]], s1, add=True).wait()

        for p in range(NCHUNK // HALF):
            pltpu.sync_copy(row_hbm.at[gid, pl.ds(p * HALF, HALF)], ridx)
            pltpu.sync_copy(col_hbm.at[gid, pl.ds(p * HALF, HALF)], cidx)

            gstart(0, gb0, s0)
            gstart(1, gb1, s2)

            def body(i, carry):
                j = 2 * i
                gwait(j, gb0, s0)
                scat(j, gb0)
                gstart(j + 2, gb0, s0)
                gwait(j + 1, gb1, s2)
                scat(j + 1, gb1)
                gstart(j + 3, gb1, s2)
                return carry

            lax.fori_loop(0, HALF // 2 - 1, body, 0)
            jt = HALF - 2
            gwait(jt, gb0, s0)
            scat(jt, gb0)
            gwait(jt + 1, gb1, s2)
            scat(jt + 1, gb1)

        plsc.subcore_barrier()

        def wb_start(t, buf, sem):
            pltpu.make_async_copy(acc.at[iov.at[t]], buf, sem).start()

        def wb_wait(t, buf, sem):
            pltpu.make_async_copy(acc.at[iov.at[t]], buf, sem).wait()

        wb_start(0, gb0, s0)
        for t in range(NWB):
            buf, sem = (gb0, s0) if t % 2 == 0 else (gb1, s2)
            nbuf, nsem = (gb1, s2) if t % 2 == 0 else (gb0, s0)
            wb_wait(t, buf, sem)
            if t + 1 < NWB:
                wb_start(t + 1, nbuf, nsem)
            pltpu.sync_copy(buf, out_hbm.at[cid, pl.ds(base + t * WB, WB)])

    return k(y, row_c, col_c, iota_c)


_R = 1000
_G = N_NODES // _R


def _dinv_block(d0_ref, d1_ref):
    deg = d0_ref[...][0] + d1_ref[...][0] + 1.0
    return lax.rsqrt(deg)


def _deg_specs():
    return [
        pl.BlockSpec((1, _R, 1), lambda i: (0, i, 0)),
        pl.BlockSpec((1, _R, 1), lambda i: (1, i, 0)),
    ]


def _ap_specs():
    return [
        pl.BlockSpec((1, _R, D), lambda i: (0, i, 0)),
        pl.BlockSpec((1, _R, D), lambda i: (1, i, 0)),
    ]


def _tc_mm1(x, w1, degp):
    def body(x_ref, w_ref, d0_ref, d1_ref, y_ref):
        dinv = _dinv_block(d0_ref, d1_ref)
        y_ref[...] = jnp.dot(x_ref[...], w_ref[...],
                             preferred_element_type=jnp.float32) * dinv

    return pl.pallas_call(
        body,
        grid=(_G,),
        in_specs=[
            pl.BlockSpec((_R, D), lambda i: (i, 0)),
            pl.BlockSpec((D, D), lambda i: (0, 0)),
        ] + _deg_specs(),
        out_specs=pl.BlockSpec((_R, D), lambda i: (i, 0)),
        out_shape=jax.ShapeDtypeStruct((N_NODES, D), jnp.float32),
    )(x, w1, degp, degp)


def _tc_mm2(ap, y1, degp, b1, w2):
    def body(a0_ref, a1_ref, y1_ref, d0_ref, d1_ref, b_ref, w_ref, y2_ref):
        dinv = _dinv_block(d0_ref, d1_ref)
        h = (a0_ref[...][0] + a1_ref[...][0] + y1_ref[...]) * dinv + b_ref[...]
        h = jnp.maximum(h, 0.0)
        y2_ref[...] = jnp.dot(h, w_ref[...],
                              preferred_element_type=jnp.float32) * dinv

    return pl.pallas_call(
        body,
        grid=(_G,),
        in_specs=_ap_specs() + [
            pl.BlockSpec((_R, D), lambda i: (i, 0)),
        ] + _deg_specs() + [
            pl.BlockSpec((1, D), lambda i: (0, 0)),
            pl.BlockSpec((D, D), lambda i: (0, 0)),
        ],
        out_specs=pl.BlockSpec((_R, D), lambda i: (i, 0)),
        out_shape=jax.ShapeDtypeStruct((N_NODES, D), jnp.float32),
    )(ap, ap, y1, degp, degp, b1, w2)


def _tc_final(ap, y2, degp, b2):
    def body(a0_ref, a1_ref, y2_ref, d0_ref, d1_ref, b_ref, o_ref):
        dinv = _dinv_block(d0_ref, d1_ref)
        h = (a0_ref[...][0] + a1_ref[...][0] + y2_ref[...]) * dinv + b_ref[...]
        o_ref[...] = jnp.maximum(h, 0.0)

    return pl.pallas_call(
        body,
        grid=(_G,),
        in_specs=_ap_specs() + [
            pl.BlockSpec((_R, D), lambda i: (i, 0)),
        ] + _deg_specs() + [
            pl.BlockSpec((1, D), lambda i: (0, 0)),
        ],
        out_specs=pl.BlockSpec((_R, D), lambda i: (i, 0)),
        out_shape=jax.ShapeDtypeStruct((N_NODES, D), jnp.float32),
    )(ap, ap, y2, degp, degp, b2)


def kernel(x, edge_index, W1, b1, W2, b2):
    ei = edge_index.astype(jnp.int32)
    row = ei[0].reshape(NW, EPT)
    col = ei[1].reshape(NW, EPT)
    npad = EPT_PAD - EPT
    row_c = jnp.concatenate(
        [row, jnp.zeros((NW, npad), jnp.int32)], axis=1).reshape(NW, NCHUNK, CHUNK)
    col_c = jnp.concatenate(
        [col, jnp.full((NW, npad), DUMMY, jnp.int32)], axis=1).reshape(NW, NCHUNK, CHUNK)

    iota_c = jnp.arange(NPAD, dtype=jnp.int32).reshape(NS, NWB, CHUNK)
    degr = _sc_degree(col_c, iota_c)
    degp = degr[:, :, :NWB, :].reshape(NC, NPAD, 1)
    b1r = b1.reshape(1, D)
    b2r = b2.reshape(1, D)

    y1 = _tc_mm1(x, W1, degp)
    a1 = _sc_scatter(y1, row_c, col_c, iota_c)
    y2 = _tc_mm2(a1, y1, degp, b1r, W2)
    a2 = _sc_scatter(y2, row_c, col_c, iota_c)
    return _tc_final(a2, y2, degp, b2r)


# speedup vs baseline: 1.0817x; 1.0006x over previous
import functools

import jax
import jax.numpy as jnp
from jax import lax
from jax.experimental import pallas as pl
from jax.experimental.pallas import tpu as pltpu
from jax.experimental.pallas import tpu_sc as plsc

N_NODES = 10000
N_EDGES = 320000
D = 128

NC = 2
NS = 16
NW = NC * NS
EPT = N_EDGES // NW
CHUNK = 128
NCHUNK = ((-(-EPT // CHUNK) + 15) // 16) * 16
EPT_PAD = NCHUNK * CHUNK
HALF = NCHUNK // 2
DUMMY = N_NODES
NPAD = 10240
RPT = NPAD // NS
WB = 128
NWB = RPT // WB
LANES = 16


def _mesh():
    return plsc.VectorSubcoreMesh(core_axis_name="c", subcore_axis_name="s")


def _sc_degree(col_c, iota_c):
    @functools.partial(
        pl.kernel,
        out_type=jax.ShapeDtypeStruct((NC, NS, 8, CHUNK), jnp.float32),
        mesh=_mesh(),
        scratch_types=[
            pltpu.VMEM((NCHUNK, CHUNK), jnp.int32),
            pltpu.VMEM((NWB, CHUNK), jnp.int32),
            pltpu.VMEM((CHUNK,), jnp.float32),
            pltpu.VMEM((CHUNK,), jnp.float32),
            pltpu.VMEM((8, CHUNK), jnp.float32),
            pltpu.VMEM((CHUNK,), jnp.int32),
            pltpu.VMEM_SHARED((NPAD,), jnp.float32),
            pltpu.SemaphoreType.DMA,
        ],
    )
    def k(col_hbm, iota_hbm, out_hbm, cidx, iov, ones_b, zb, wb8, ibuf, deg_sh, sem):
        cid = lax.axis_index("c")
        sid = lax.axis_index("s")
        gid = cid * NS + sid
        pltpu.sync_copy(col_hbm.at[gid], cidx)
        pltpu.sync_copy(iota_hbm.at[sid], iov)

        ones = jnp.ones((LANES,), jnp.float32)
        zeros = jnp.zeros((LANES,), jnp.float32)
        for kk in range(CHUNK // LANES):
            ones_b[pl.ds(kk * LANES, LANES)] = ones
            zb[pl.ds(kk * LANES, LANES)] = zeros

        def load_ibuf(src_ref, j):
            for kk in range(CHUNK // LANES):
                ibuf[pl.ds(kk * LANES, LANES)] = src_ref[j, pl.ds(kk * LANES, LANES)]

        for t in range(NWB):
            load_ibuf(iov, t)
            pltpu.async_copy(zb, deg_sh.at[ibuf], sem).wait()
        plsc.subcore_barrier()

        def body(j, carry):
            load_ibuf(cidx, j)
            pltpu.async_copy(ones_b, deg_sh.at[ibuf], sem, add=True).wait()
            return carry

        lax.fori_loop(0, NCHUNK, body, 0)

        plsc.subcore_barrier()
        for t in range(NWB):
            load_ibuf(iov, t)
            pltpu.async_copy(deg_sh.at[ibuf], zb, sem).wait()
            for kk in range(CHUNK // LANES):
                wb8[t, pl.ds(kk * LANES, LANES)] = zb[pl.ds(kk * LANES, LANES)]
        pltpu.sync_copy(wb8, out_hbm.at[cid, sid])

    return k(col_c, iota_c)


def _sc_scatter(y, row_c, col_c, iota_c):
    @functools.partial(
        pl.kernel,
        out_type=jax.ShapeDtypeStruct((NC, NPAD, D), jnp.float32),
        mesh=_mesh(),
        scratch_types=[
            pltpu.VMEM((HALF, CHUNK), jnp.int32),
            pltpu.VMEM((HALF, CHUNK), jnp.int32),
            pltpu.VMEM((NWB, CHUNK), jnp.int32),
            pltpu.VMEM((CHUNK,), jnp.int32),
            pltpu.VMEM((CHUNK, D), jnp.float32),
            pltpu.VMEM((CHUNK, D), jnp.float32),
            pltpu.VMEM_SHARED((NPAD, D), jnp.float32),
            pltpu.SemaphoreType.DMA,
            pltpu.SemaphoreType.DMA,
            pltpu.SemaphoreType.DMA,
        ],
    )
    def k(y_hbm, row_hbm, col_hbm, iota_hbm, out_hbm,
          ridx, cidx, iov, ibuf, gb0, gb1, acc, s0, s1, s2):
        cid = lax.axis_index("c")
        sid = lax.axis_index("s")
        gid = cid * NS + sid
        pltpu.sync_copy(iota_hbm.at[sid], iov)

        def load_ibuf(src_ref, j):
            for kk in range(CHUNK // LANES):
                ibuf[pl.ds(kk * LANES, LANES)] = src_ref[j, pl.ds(kk * LANES, LANES)]

        def zero(i, carry):
            for kk in range(D // LANES):
                gb0[i, pl.ds(kk * LANES, LANES)] = jnp.zeros((LANES,), jnp.float32)
            return carry

        lax.fori_loop(0, CHUNK, zero, 0)

        base = sid * RPT
        for t in range(NWB):
            load_ibuf(iov, t)
            pltpu.async_copy(gb0, acc.at[ibuf], s0).wait()
        plsc.subcore_barrier()

        def gstart(j, buf, sem):
            pltpu.make_async_copy(y_hbm.at[ridx.at[j]], buf, sem).start()

        def gwait(j, buf, sem):
            pltpu.make_async_copy(y_hbm.at[ridx.at[j]], buf, sem).wait()

        def scat(j, buf):
            pltpu.async_copy(buf, acc.at[cidx.at[j]], s1, add=True).wait()

        for p in range(NCHUNK // HALF):
            pltpu.sync_copy(row_hbm.at[gid, pl.ds(p * HALF, HALF)], ridx)
            pltpu.sync_copy(col_hbm.at[gid, pl.ds(p * HALF, HALF)], cidx)

            gstart(0, gb0, s0)
            gstart(1, gb1, s2)

            def body(i, carry):
                j = 2 * i
                gwait(j, gb0, s0)
                scat(j, gb0)
                gstart(j + 2, gb0, s0)
                gwait(j + 1, gb1, s2)
                scat(j + 1, gb1)
                gstart(j + 3, gb1, s2)
                return carry

            lax.fori_loop(0, HALF // 2 - 1, body, 0)
            jt = HALF - 2
            gwait(jt, gb0, s0)
            scat(jt, gb0)
            gwait(jt + 1, gb1, s2)
            scat(jt + 1, gb1)

        plsc.subcore_barrier()

        def wb_start(t, buf, sem):
            pltpu.make_async_copy(acc.at[iov.at[t]], buf, sem).start()

        def wb_wait(t, buf, sem):
            pltpu.make_async_copy(acc.at[iov.at[t]], buf, sem).wait()

        wb_start(0, gb0, s0)
        for t in range(NWB):
            buf, sem = (gb0, s0) if t % 2 == 0 else (gb1, s2)
            nbuf, nsem = (gb1, s2) if t % 2 == 0 else (gb0, s0)
            wb_wait(t, buf, sem)
            if t + 1 < NWB:
                wb_start(t + 1, nbuf, nsem)
            pltpu.sync_copy(buf, out_hbm.at[cid, pl.ds(base + t * WB, WB)])

    return k(y, row_c, col_c, iota_c)


_R = 1000
_G = N_NODES // _R


def _dinv_block(d0_ref, d1_ref):
    deg = d0_ref[...][0] + d1_ref[...][0] + 1.0
    return lax.rsqrt(deg)


def _deg_specs():
    return [
        pl.BlockSpec((1, _R, 1), lambda i: (0, i, 0)),
        pl.BlockSpec((1, _R, 1), lambda i: (1, i, 0)),
    ]


def _ap_specs():
    return [
        pl.BlockSpec((1, _R, D), lambda i: (0, i, 0)),
        pl.BlockSpec((1, _R, D), lambda i: (1, i, 0)),
    ]


def _tc_mm1(x, w1, degp):
    def body(x_ref, w_ref, d0_ref, d1_ref, y_ref):
        dinv = _dinv_block(d0_ref, d1_ref)
        y_ref[...] = jnp.dot(x_ref[...], w_ref[...],
                             preferred_element_type=jnp.float32) * dinv

    return pl.pallas_call(
        body,
        grid=(_G,),
        in_specs=[
            pl.BlockSpec((_R, D), lambda i: (i, 0)),
            pl.BlockSpec((D, D), lambda i: (0, 0)),
        ] + _deg_specs(),
        out_specs=pl.BlockSpec((_R, D), lambda i: (i, 0)),
        out_shape=jax.ShapeDtypeStruct((N_NODES, D), jnp.float32),
    )(x, w1, degp, degp)


def _tc_mm2(ap, y1, degp, b1, w2):
    def body(a0_ref, a1_ref, y1_ref, d0_ref, d1_ref, b_ref, w_ref, y2_ref):
        dinv = _dinv_block(d0_ref, d1_ref)
        h = (a0_ref[...][0] + a1_ref[...][0] + y1_ref[...]) * dinv + b_ref[...]
        h = jnp.maximum(h, 0.0)
        y2_ref[...] = jnp.dot(h, w_ref[...],
                              preferred_element_type=jnp.float32) * dinv

    return pl.pallas_call(
        body,
        grid=(_G,),
        in_specs=_ap_specs() + [
            pl.BlockSpec((_R, D), lambda i: (i, 0)),
        ] + _deg_specs() + [
            pl.BlockSpec((1, D), lambda i: (0, 0)),
            pl.BlockSpec((D, D), lambda i: (0, 0)),
        ],
        out_specs=pl.BlockSpec((_R, D), lambda i: (i, 0)),
        out_shape=jax.ShapeDtypeStruct((N_NODES, D), jnp.float32),
    )(ap, ap, y1, degp, degp, b1, w2)


def _tc_final(ap, y2, degp, b2):
    def body(a0_ref, a1_ref, y2_ref, d0_ref, d1_ref, b_ref, o_ref):
        dinv = _dinv_block(d0_ref, d1_ref)
        h = (a0_ref[...][0] + a1_ref[...][0] + y2_ref[...]) * dinv + b_ref[...]
        o_ref[...] = jnp.maximum(h, 0.0)

    return pl.pallas_call(
        body,
        grid=(_G,),
        in_specs=_ap_specs() + [
            pl.BlockSpec((_R, D), lambda i: (i, 0)),
        ] + _deg_specs() + [
            pl.BlockSpec((1, D), lambda i: (0, 0)),
        ],
        out_specs=pl.BlockSpec((_R, D), lambda i: (i, 0)),
        out_shape=jax.ShapeDtypeStruct((N_NODES, D), jnp.float32),
    )(ap, ap, y2, degp, degp, b2)


def kernel(x, edge_index, W1, b1, W2, b2):
    ei = edge_index.astype(jnp.int32)
    row = ei[0].reshape(NW, EPT)
    col = ei[1].reshape(NW, EPT)
    npad = EPT_PAD - EPT
    row_c = jnp.concatenate(
        [row, jnp.zeros((NW, npad), jnp.int32)], axis=1).reshape(NW, NCHUNK, CHUNK)
    col_c = jnp.concatenate(
        [col, jnp.full((NW, npad), DUMMY, jnp.int32)], axis=1).reshape(NW, NCHUNK, CHUNK)

    iota_c = jnp.arange(NPAD, dtype=jnp.int32).reshape(NS, NWB, CHUNK)
    degr = _sc_degree(col_c, iota_c)
    degp = degr[:, :, :NWB, :].reshape(NC, NPAD, 1)
    b1r = b1.reshape(1, D)
    b2r = b2.reshape(1, D)

    y1 = _tc_mm1(x, W1, degp)
    a1 = _sc_scatter(y1, row_c, col_c, iota_c)
    y2 = _tc_mm2(a1, y1, degp, b1r, W2)
    a2 = _sc_scatter(y2, row_c, col_c, iota_c)
    return _tc_final(a2, y2, degp, b2r)
